# Initial kernel scaffold; baseline (speedup 1.0000x reference)
import jax
import jax.numpy as jnp
from jax.experimental import pallas as pl


def kernel(X, table):
    raise NotImplementedError("write your pallas kernel here")


# speedup vs baseline: 1.6843x; 1.6843x over previous
import functools

import jax
import jax.numpy as jnp
from jax import lax
from jax.experimental import pallas as pl
from jax.experimental.pallas import tpu as pltpu
from jax.experimental.pallas import tpu_sc as plsc

VOCAB = 1000000
EMBED_DIM = 64
BATCH = 16384
HIST = 50

_INFO = plsc.get_sparse_core_info()
_NC = _INFO.num_cores
_NS = _INFO.num_subcores
_NW = _NC * _NS

_TOTAL = BATCH * HIST
_PER_W = _TOTAL // _NW
_CSZ = 128
_CH = _PER_W // _CSZ


def _make_gather():
    mesh = plsc.VectorSubcoreMesh(core_axis_name="c", subcore_axis_name="s")

    @functools.partial(
        pl.kernel,
        mesh=mesh,
        compiler_params=pltpu.CompilerParams(use_tc_tiling_on_sc=False),
        out_type=jax.ShapeDtypeStruct((_TOTAL, EMBED_DIM), jnp.float32),
        scratch_types=[
            pltpu.VMEM((_CH, _CSZ), jnp.int32),
            pltpu.VMEM((_CSZ, EMBED_DIM), jnp.float32),
            pltpu.VMEM((_CSZ, EMBED_DIM), jnp.float32),
            pltpu.SemaphoreType.DMA,
            pltpu.SemaphoreType.DMA,
        ],
    )
    def gather_kernel(idx_hbm, table_hbm, out_hbm, idx_v, buf0, buf1, sem0, sem1):
        wid = lax.axis_index("s") * _NC + lax.axis_index("c")
        base = wid * _PER_W
        pltpu.sync_copy(idx_hbm.at[wid], idx_v)

        bufs = (buf0, buf1)
        sems = (sem0, sem1)

        def chunk(j, buf, sem):
            cp = pltpu.async_copy(table_hbm.at[idx_v.at[j]], buf, sem)
            cp.wait()
            pltpu.sync_copy(buf, out_hbm.at[pl.ds(base + j * _CSZ, _CSZ)])

        def body(jj, _):
            for b in range(2):
                chunk(jj * 2 + b, bufs[b], sems[b])
            return _

        lax.fori_loop(0, _CH // 2, body, 0, unroll=False)

    return gather_kernel


_gather = _make_gather()


def kernel(X, table):
    idx = X.astype(jnp.int32).reshape(_NW, _CH, _CSZ)
    out = _gather(idx, table)
    return out.reshape(BATCH, HIST, EMBED_DIM)


# speedup vs baseline: 1.8764x; 1.1141x over previous
import functools

import jax
import jax.numpy as jnp
from jax import lax
from jax.experimental import pallas as pl
from jax.experimental.pallas import tpu as pltpu
from jax.experimental.pallas import tpu_sc as plsc

VOCAB = 1000000
EMBED_DIM = 64
BATCH = 16384
HIST = 50

_INFO = plsc.get_sparse_core_info()
_NC = _INFO.num_cores
_NS = _INFO.num_subcores
_NW = _NC * _NS

_TOTAL = BATCH * HIST
_PER_W = _TOTAL // _NW
_CSZ = 128
_CH = _PER_W // _CSZ
_NBUF = 8


def _make_gather():
    mesh = plsc.VectorSubcoreMesh(core_axis_name="c", subcore_axis_name="s")

    @functools.partial(
        pl.kernel,
        mesh=mesh,
        compiler_params=pltpu.CompilerParams(use_tc_tiling_on_sc=False),
        out_type=jax.ShapeDtypeStruct((_TOTAL, EMBED_DIM), jnp.float32),
        scratch_types=[
            pltpu.VMEM((_CH, _CSZ), jnp.int32),
            [pltpu.VMEM((_CSZ, EMBED_DIM), jnp.float32)] * _NBUF,
            [pltpu.SemaphoreType.DMA] * _NBUF,
            [pltpu.SemaphoreType.DMA] * _NBUF,
        ],
    )
    def gather_kernel(idx_hbm, table_hbm, out_hbm, idx_v, bufs, gsems, wsems):
        wid = lax.axis_index("s") * _NC + lax.axis_index("c")
        base = wid * _PER_W
        pltpu.sync_copy(idx_hbm.at[wid], idx_v)

        def out_at(j):
            return out_hbm.at[pl.ds(base + j * _CSZ, _CSZ)]

        def start_gather(j, b):
            pltpu.async_copy(table_hbm.at[idx_v.at[j]], bufs[b], gsems[b])

        def finish_gather(j, b):
            pltpu.make_async_copy(table_hbm.at[idx_v.at[j]], bufs[b], gsems[b]).wait()

        def start_write(j, b):
            pltpu.async_copy(bufs[b], out_at(j), wsems[b])

        def finish_write(j, b):
            pltpu.make_async_copy(bufs[b], out_at(j), wsems[b]).wait()

        def body(g, carry):
            for b in range(_NBUF):
                j = g * _NBUF + b
                @pl.when(g > 0)
                def _(b=b, j=j):
                    finish_write(j - _NBUF, b)

                start_gather(j, b)

                kb = (b + 1) % _NBUF
                if b == _NBUF - 1:
                    finish_gather(j - _NBUF + 1, kb)
                    start_write(j - _NBUF + 1, kb)
                else:
                    @pl.when(g > 0)
                    def _(b=b, j=j, kb=kb):
                        finish_gather(j - _NBUF + 1, kb)
                        start_write(j - _NBUF + 1, kb)
            return carry

        lax.fori_loop(0, _CH // _NBUF, body, 0, unroll=False)

        for i in range(1, _NBUF):
            k = _CH - _NBUF + i
            finish_gather(k, i)
            start_write(k, i)
        for b in range(_NBUF):
            finish_write(_CH - _NBUF + b, b)

    return gather_kernel


_gather = _make_gather()


def kernel(X, table):
    idx = X.astype(jnp.int32).reshape(_NW, _CH, _CSZ)
    out = _gather(idx, table)
    return out.reshape(BATCH, HIST, EMBED_DIM)
